# baseline (speedup 1.0000x reference)
import jax
import jax.numpy as jnp
from jax.experimental import pallas as pl
from jax.experimental.pallas import tpu as pltpu

DT = 15
B, H, W = 4, 512, 512


def _body(x_ref, y_ref, m_ref, o_ref, z0_ref, z1_ref):
    d = y_ref[...] - x_ref[...]
    o_ref[...] = jnp.where(m_ref[...] != 0, d, 0.0)
    z0_ref[...] = jnp.zeros_like(z0_ref)
    z1_ref[...] = jnp.zeros_like(z1_ref)


def kernel(x, ylr, msk_lr):
    m8 = msk_lr.astype(jnp.int8)
    bt = 5
    grid = (B, DT // bt)
    spec = pl.BlockSpec((1, bt, H, W), lambda b, t: (b, t, 0, 0))
    oshape = jax.ShapeDtypeStruct((B, DT, H, W), jnp.float32)
    out, z0, z1 = pl.pallas_call(
        _body,
        grid=grid,
        in_specs=[spec, spec, spec],
        out_specs=[spec, spec, spec],
        out_shape=[oshape, oshape, oshape],
        compiler_params=pltpu.CompilerParams(
            dimension_semantics=("arbitrary", "arbitrary"),
            allow_input_fusion=(False, False, True),
        ),
    )(x, ylr, m8)
    return out, z0, z1


# speedup vs baseline: 1.0146x; 1.0146x over previous
import jax
import jax.numpy as jnp
from jax.experimental import pallas as pl
from jax.experimental.pallas import tpu as pltpu

DT = 15
B, H, W = 4, 512, 512


def _body(x_ref, y_ref, m_ref, o_ref, z0_ref, z1_ref):
    d = y_ref[...] - x_ref[...]
    o_ref[...] = jnp.where(m_ref[...] != 0, d, 0.0)
    z0_ref[...] = jnp.zeros_like(z0_ref)
    z1_ref[...] = jnp.zeros_like(z1_ref)


def kernel(x, ylr, msk_lr):
    m8 = msk_lr.astype(jnp.int8)
    bt = 1
    grid = (B, DT // bt)
    spec = pl.BlockSpec((1, bt, H, W), lambda b, t: (b, t, 0, 0))
    oshape = jax.ShapeDtypeStruct((B, DT, H, W), jnp.float32)
    out, z0, z1 = pl.pallas_call(
        _body,
        grid=grid,
        in_specs=[spec, spec, spec],
        out_specs=[spec, spec, spec],
        out_shape=[oshape, oshape, oshape],
        compiler_params=pltpu.CompilerParams(
            dimension_semantics=("arbitrary", "arbitrary"),
            allow_input_fusion=(False, False, True),
        ),
    )(x, ylr, m8)
    return out, z0, z1
